# baseline (speedup 1.0000x reference)
import functools

import jax
import jax.numpy as jnp
from jax import lax
from jax.experimental import pallas as pl
from jax.experimental.pallas import tpu as pltpu
from jax.experimental.pallas import tpu_sc as plsc

_CHUNK = 128


def _sc_gather_body(nc, n_chunks, b_per_w,
                    act_tab, res_tab, aidx, ridx, act_out, res_out,
                    aidx_v, ridx_v, arows_v, rrows_v, sem_a, sem_r):
    wid = lax.axis_index("s") * nc + lax.axis_index("c")
    pltpu.sync_copy(aidx.at[wid], aidx_v)
    pltpu.sync_copy(ridx.at[wid], ridx_v)
    copies = []
    for j in range(n_chunks):
        copies.append(pltpu.async_copy(
            act_tab.at[aidx_v.at[j]], arows_v.at[pl.ds(j * _CHUNK, _CHUNK)],
            sem_a))
        copies.append(pltpu.async_copy(
            res_tab.at[ridx_v.at[j]], rrows_v.at[pl.ds(j * _CHUNK, _CHUNK)],
            sem_r))
    for c in copies:
        c.wait()
    pltpu.sync_copy(arows_v, act_out.at[pl.ds(wid * b_per_w, b_per_w)])
    pltpu.sync_copy(rrows_v, res_out.at[pl.ds(wid * b_per_w, b_per_w)])


def _sc_gather(act_table, res_table, activities, resources):
    n = activities.shape[0]
    h = act_table.shape[1]
    info = plsc.get_sparse_core_info()
    nc, ns = info.num_cores, info.num_subcores
    nw = nc * ns
    b_per_w = n // nw
    assert b_per_w * nw == n and b_per_w % _CHUNK == 0
    n_chunks = b_per_w // _CHUNK
    aidx3 = activities.reshape(nw, n_chunks, _CHUNK)
    ridx3 = resources.reshape(nw, n_chunks, _CHUNK)
    mesh = plsc.VectorSubcoreMesh(core_axis_name="c", subcore_axis_name="s")
    f = pl.kernel(
        functools.partial(_sc_gather_body, nc, n_chunks, b_per_w),
        compiler_params=pltpu.CompilerParams(use_tc_tiling_on_sc=False),
        out_type=(jax.ShapeDtypeStruct((n, h), jnp.float32),
                  jax.ShapeDtypeStruct((n, h), jnp.float32)),
        mesh=mesh,
        scratch_types=[
            pltpu.VMEM((n_chunks, _CHUNK), jnp.int32),
            pltpu.VMEM((n_chunks, _CHUNK), jnp.int32),
            pltpu.VMEM((b_per_w, h), jnp.float32),
            pltpu.VMEM((b_per_w, h), jnp.float32),
            pltpu.SemaphoreType.DMA,
            pltpu.SemaphoreType.DMA,
        ],
    )
    return f(act_table, res_table, aidx3, ridx3)


def _tc_dense_body(act_ref, res_ref, num_ref, aid_ref, rid_ref,
                   nlg_ref, nlb_ref, w1_ref, b1_ref, mlg_ref, mlb_ref,
                   wg_ref, bg_ref, wb_ref, bb_ref, wpc_ref, wpn_ref, bp_ref,
                   plg_ref, plb_ref, out_ref):
    eps = 1e-5
    num = num_ref[...]
    nf = jnp.log(1.0 + jnp.maximum(num, 0.0))
    mu = jnp.mean(nf, axis=-1, keepdims=True)
    var = jnp.mean((nf - mu) ** 2, axis=-1, keepdims=True)
    nf = (nf - mu) * lax.rsqrt(var + eps) * nlg_ref[...] + nlb_ref[...]
    hid = jnp.dot(nf, w1_ref[...], preferred_element_type=jnp.float32)
    hid = jnp.maximum(hid + b1_ref[...], 0.0)
    mu = jnp.mean(hid, axis=-1, keepdims=True)
    var = jnp.mean((hid - mu) ** 2, axis=-1, keepdims=True)
    num_emb = (hid - mu) * lax.rsqrt(var + eps) * mlg_ref[...] + mlb_ref[...]
    g_in = jnp.dot(num_emb, wg_ref[...], preferred_element_type=jnp.float32)
    gamma = 1.0 / (1.0 + jnp.exp(-(g_in + bg_ref[...])))
    beta = jnp.dot(num_emb, wb_ref[...],
                   preferred_element_type=jnp.float32) + bb_ref[...]
    cat = jnp.concatenate([act_ref[...], res_ref[...]], axis=-1)
    cat_mod = cat * gamma + beta
    is_pad = (aid_ref[...] == 0) & (rid_ref[...] == 0)
    cat_mod = jnp.where(is_pad, 0.0, cat_mod)
    num_emb = jnp.where(is_pad, 0.0, num_emb)
    comb = (jnp.dot(cat_mod, wpc_ref[...], preferred_element_type=jnp.float32)
            + jnp.dot(num_emb, wpn_ref[...], preferred_element_type=jnp.float32)
            + bp_ref[...])
    comb = jnp.maximum(comb, 0.0)
    mu = jnp.mean(comb, axis=-1, keepdims=True)
    var = jnp.mean((comb - mu) ** 2, axis=-1, keepdims=True)
    out_ref[...] = (comb - mu) * lax.rsqrt(var + eps) * plg_ref[...] + plb_ref[...]


def kernel(activities, resources, num_arr, act_table, res_table,
           num_ln_g, num_ln_b, W1, b1, mlp_ln_g, mlp_ln_b,
           Wg, bg, Wb, bb, Wp, bp, proj_ln_g, proj_ln_b):
    n = activities.shape[0]
    h = act_table.shape[1]
    d = W1.shape[1]
    f = num_arr.shape[1]
    acts = activities.astype(jnp.int32)
    ress = resources.astype(jnp.int32)

    act_emb, res_emb = _sc_gather(act_table, res_table, acts, ress)

    bn = 1024
    nblk = n // bn
    row_spec = lambda w: pl.BlockSpec((bn, w), lambda i: (i, 0))
    full_spec = lambda s: pl.BlockSpec(s, lambda i: tuple(0 for _ in s))
    out = pl.pallas_call(
        _tc_dense_body,
        grid=(nblk,),
        in_specs=[
            row_spec(h), row_spec(h), row_spec(f), row_spec(1), row_spec(1),
            full_spec((1, f)), full_spec((1, f)),
            full_spec((f, d)), full_spec((1, d)),
            full_spec((1, d)), full_spec((1, d)),
            full_spec((d, d)), full_spec((1, d)),
            full_spec((d, d)), full_spec((1, d)),
            full_spec((d, d)), full_spec((d, d)), full_spec((1, d)),
            full_spec((1, d)), full_spec((1, d)),
        ],
        out_specs=row_spec(d),
        out_shape=jax.ShapeDtypeStruct((n, d), jnp.float32),
    )(
        act_emb, res_emb, num_arr,
        acts.reshape(n, 1), ress.reshape(n, 1),
        num_ln_g.reshape(1, f), num_ln_b.reshape(1, f),
        W1, b1.reshape(1, d),
        mlp_ln_g.reshape(1, d), mlp_ln_b.reshape(1, d),
        Wg, bg.reshape(1, d),
        Wb, bb.reshape(1, d),
        Wp[:d], Wp[d:], bp.reshape(1, d),
        proj_ln_g.reshape(1, d), proj_ln_b.reshape(1, d),
    )
    return out
